# Initial kernel scaffold; baseline (speedup 1.0000x reference)
import jax
import jax.numpy as jnp
from jax.experimental import pallas as pl


def kernel(x, edge_index, batch, W1l, b1, W1r, W2l, b2, W2r, Wlin, blin):
    raise NotImplementedError("write your pallas kernel here")


# speedup vs baseline: 3.5179x; 3.5179x over previous
import functools

import jax
import jax.numpy as jnp
from jax import lax
from jax.experimental import pallas as pl
from jax.experimental.pallas import tpu as pltpu
from jax.experimental.pallas import tpu_sc as plsc

N_NODES = 10000
N_EDGES = 320000
D_IN = 128
D_HID = 256
D_OUT = 12
N_GRAPHS = 64

NC = 2
NS = 16
LANES = 16

EROWS = 2560
E_PAD = EROWS * 128
RPT = EROWS // NS
NACC = 10112
ZROWS = NACC // NS


IBL = 32
NCHUNK = RPT // IBL


def _agg_body(t0, t1, src2, dst2, z2, sums_out, sidx, didx, rb, acc, gsem):
    cid = lax.axis_index("c")
    sid = lax.axis_index("s")

    pltpu.sync_copy(z2.at[pl.ds(sid * ZROWS, ZROWS)],
                    acc.at[pl.ds(sid * ZROWS, ZROWS)])
    plsc.subcore_barrier()

    def chunk(c, _):
        base = sid * RPT + c * IBL
        pltpu.sync_copy(src2.at[pl.ds(base, IBL)], sidx)
        pltpu.sync_copy(dst2.at[pl.ds(base, IBL)], didx)

        def step(j, _):
            @pl.when(cid == 0)
            def _():
                pltpu.async_copy(t0.at[sidx.at[j]], rb, gsem).wait()

            @pl.when(cid == 1)
            def _():
                pltpu.async_copy(t1.at[sidx.at[j]], rb, gsem).wait()

            pltpu.sync_copy(rb, acc.at[didx.at[j]], add=True)
            return 0

        lax.fori_loop(0, IBL, step, 0)
        return 0

    lax.fori_loop(0, NCHUNK, chunk, 0)
    plsc.subcore_barrier()

    pltpu.sync_copy(acc.at[pl.ds(sid * ZROWS, ZROWS)],
                    sums_out.at[cid, pl.ds(sid * ZROWS, ZROWS)])


HRPT = EROWS // (NC * NS)


def _agg_counts_body(t, src2, dst2, z2, z1, sums_out, cnt_out,
                     sidx, didx, rb, onesv, acc, accc, gsem):
    cid = lax.axis_index("c")
    sid = lax.axis_index("s")

    pltpu.sync_copy(z2.at[pl.ds(sid * ZROWS, ZROWS)],
                    acc.at[pl.ds(sid * ZROWS, ZROWS)])

    @pl.when(sid == 0)
    def _():
        pltpu.sync_copy(z1, accc)

    for i in range(128 // LANES):
        onesv[pl.ds(i * LANES, LANES)] = jnp.ones((LANES,), jnp.float32)

    base = (cid * NS + sid) * HRPT
    pltpu.sync_copy(src2.at[pl.ds(base, HRPT)], sidx)
    pltpu.sync_copy(dst2.at[pl.ds(base, HRPT)], didx)
    plsc.subcore_barrier()

    def step(j, _):
        pltpu.async_copy(t.at[sidx.at[j]], rb, gsem).wait()
        pltpu.sync_copy(rb, acc.at[didx.at[j]], add=True)
        pltpu.sync_copy(onesv, accc.at[didx.at[j]], add=True)
        return 0

    lax.fori_loop(0, HRPT, step, 0)
    plsc.subcore_barrier()

    pltpu.sync_copy(acc.at[pl.ds(sid * ZROWS, ZROWS)],
                    sums_out.at[cid, pl.ds(sid * ZROWS, ZROWS)])

    @pl.when(sid == 0)
    def _():
        pltpu.sync_copy(accc, cnt_out.at[pl.ds(cid * NACC, NACC)])


def _make_agg(C, with_counts):
    mesh = plsc.VectorSubcoreMesh(core_axis_name="c", subcore_axis_name="s")
    if with_counts:
        out_type = (jax.ShapeDtypeStruct((NC, NACC, C), jnp.float32),
                    jax.ShapeDtypeStruct((NC * NACC,), jnp.float32))
        scratch = [
            pltpu.VMEM((HRPT, 128), jnp.int32),
            pltpu.VMEM((HRPT, 128), jnp.int32),
            pltpu.VMEM((128, C), jnp.float32),
            pltpu.VMEM((128,), jnp.float32),
            pltpu.VMEM_SHARED((NACC, C), jnp.float32),
            pltpu.VMEM_SHARED((NACC,), jnp.float32),
            pltpu.SemaphoreType.DMA,
        ]
        return pl.kernel(_agg_counts_body, out_type=out_type, mesh=mesh,
                         scratch_types=scratch)
    out_type = jax.ShapeDtypeStruct((NC, NACC, C), jnp.float32)
    scratch = [
        pltpu.VMEM((IBL, 128), jnp.int32),
        pltpu.VMEM((IBL, 128), jnp.int32),
        pltpu.VMEM((128, C), jnp.float32),
        pltpu.VMEM_SHARED((NACC, C), jnp.float32),
        pltpu.SemaphoreType.DMA,
    ]
    return pl.kernel(_agg_body, out_type=out_type, mesh=mesh,
                     scratch_types=scratch)


ROWS_TC = 1000
GRID_TC = N_NODES // ROWS_TC


def _dense1_body(s0, s1, c0, c1, x, wl, wr, b, oa, ob):
    inv = 1.0 / jnp.maximum(c0[...] + c1[...], 1.0)
    mean = (s0[...] + s1[...]) * inv
    h = jnp.dot(mean, wl[...], preferred_element_type=jnp.float32)
    h += jnp.dot(x[...], wr[...], preferred_element_type=jnp.float32)
    h = jnp.maximum(h + b[...], 0.0)
    oa[...] = h[:, :D_IN]
    ob[...] = h[:, D_IN:]


def _dense2_body(s0, s1, c0, c1, h1a, h1b, bat, wl, wr, b, wlin, blin,
                 out, pooled):
    i = pl.program_id(0)

    @pl.when(i == 0)
    def _():
        pooled[...] = jnp.zeros_like(pooled)

    inv = 1.0 / jnp.maximum(c0[...] + c1[...], 1.0)
    mean = jnp.concatenate([s0[...], s1[...]], axis=1) * inv
    h1 = jnp.concatenate([h1a[...], h1b[...]], axis=1)
    h = jnp.dot(mean, wl[...], preferred_element_type=jnp.float32)
    h += jnp.dot(h1, wr[...], preferred_element_type=jnp.float32)
    h = jnp.maximum(h + b[...], 0.0)
    oh = jnp.equal(
        bat[...],
        lax.broadcasted_iota(jnp.int32, (ROWS_TC, N_GRAPHS), 1),
    ).astype(jnp.float32)
    pooled[...] += lax.dot_general(oh, h, (((0,), (0,)), ((), ())),
                                   preferred_element_type=jnp.float32)

    @pl.when(i == GRID_TC - 1)
    def _():
        out[...] = (jnp.dot(pooled[...], wlin[...],
                            preferred_element_type=jnp.float32) + blin[...])


def _row_spec(cols):
    return pl.BlockSpec((ROWS_TC, cols), lambda i: (i, 0))


def _full_spec(r, c):
    return pl.BlockSpec((r, c), lambda i: (0, 0))


_dense1 = pl.pallas_call(
    _dense1_body,
    grid=(GRID_TC,),
    in_specs=[
        _row_spec(D_IN), _row_spec(D_IN), _row_spec(1), _row_spec(1),
        _row_spec(D_IN),
        _full_spec(D_IN, D_HID), _full_spec(D_IN, D_HID), _full_spec(1, D_HID),
    ],
    out_specs=[_row_spec(D_IN), _row_spec(D_IN)],
    out_shape=[jax.ShapeDtypeStruct((N_NODES, D_IN), jnp.float32),
               jax.ShapeDtypeStruct((N_NODES, D_IN), jnp.float32)],
)

_dense2 = pl.pallas_call(
    _dense2_body,
    grid=(GRID_TC,),
    in_specs=[
        _row_spec(D_IN), _row_spec(D_IN), _row_spec(1), _row_spec(1),
        _row_spec(D_IN), _row_spec(D_IN), _row_spec(1),
        _full_spec(D_HID, D_HID), _full_spec(D_HID, D_HID),
        _full_spec(1, D_HID), _full_spec(D_HID, 128), _full_spec(1, 128),
    ],
    out_specs=_full_spec(N_GRAPHS, 128),
    out_shape=jax.ShapeDtypeStruct((N_GRAPHS, 128), jnp.float32),
    scratch_shapes=[pltpu.VMEM((N_GRAPHS, D_HID), jnp.float32)],
)

_agg1 = _make_agg(128, with_counts=True)
_agg128 = _make_agg(128, with_counts=False)


@jax.jit
def kernel(x, edge_index, batch, W1l, b1, W1r, W2l, b2, W2r, Wlin, blin):
    src = edge_index[0]
    dst = edge_index[1]
    pad = E_PAD - N_EDGES
    src2 = jnp.concatenate([src, jnp.zeros((pad,), jnp.int32)]).reshape(EROWS, 128)
    dst2 = jnp.concatenate([dst, jnp.full((pad,), N_NODES, jnp.int32)]).reshape(EROWS, 128)

    z128 = jnp.zeros((NACC, 128), jnp.float32)
    z1 = jnp.zeros((NACC,), jnp.float32)

    sums1, cnt = _agg1(x, src2, dst2, z128, z1)
    cnt = cnt.reshape(NC, NACC)
    c0 = cnt[0].reshape(NACC, 1)
    c1 = cnt[1].reshape(NACC, 1)

    h1a, h1b = _dense1(sums1[0], sums1[1], c0, c1, x, W1l, W1r,
                       b1.reshape(1, D_HID))

    sums2 = _agg128(h1a, h1b, src2, dst2, z128)

    outp = _dense2(sums2[0], sums2[1], c0, c1, h1a, h1b,
                   batch.reshape(N_NODES, 1).astype(jnp.int32),
                   W2l, W2r, b2.reshape(1, D_HID),
                   jnp.pad(Wlin, ((0, 0), (0, 128 - D_OUT))),
                   jnp.pad(blin, (0, 128 - D_OUT)).reshape(1, 128))
    return outp[:, :D_OUT]
